# baseline (speedup 1.0000x reference)
import functools

import jax
import jax.numpy as jnp
from jax import lax
from jax.experimental import pallas as pl
from jax.experimental.pallas import tpu as pltpu
from jax.experimental.pallas import tpu_sc as plsc

_DIM = 1024
_B = 4
_S = 8192
_SCALE = float(_DIM) ** (-0.5)

_NC = 2
_NS = 16
_NW = _NC * _NS
_L = 16

_ROWS_PER_W = _S // _NW
_CHUNKS = (16,) * 16
_MAX_CHUNK = max(_CHUNKS)
_OFFS = tuple(sum(_CHUNKS[:k]) for k in range(len(_CHUNKS)))
_N_CHUNKS = len(_CHUNKS)


def _make_sc_broadcast():
    mesh = plsc.VectorSubcoreMesh(core_axis_name="c", subcore_axis_name="s")

    @functools.partial(
        pl.kernel,
        mesh=mesh,
        out_type=jax.ShapeDtypeStruct((_B, _S, _DIM), jnp.float32),
        scratch_types=[
            pltpu.VMEM((_MAX_CHUNK, _DIM), jnp.float32),
            pltpu.VMEM((_MAX_CHUNK, _DIM), jnp.float32),
            pltpu.SemaphoreType.DMA,
            pltpu.SemaphoreType.DMA,
            pltpu.SemaphoreType.DMA,
            pltpu.SemaphoreType.DMA,
        ],
    )
    def sc_broadcast(table_hbm, out_hbm, buf0, buf1, si0, si1, so0, so1):
        bufs = (buf0, buf1)
        sem_in = (si0, si1)
        sem_out = (so0, so1)
        wid = lax.axis_index("s") * _NC + lax.axis_index("c")
        base = wid * _ROWS_PER_W

        def start_in(i):
            cr = _CHUNKS[i]
            return pltpu.async_copy(
                table_hbm.at[pl.ds(base + _OFFS[i], cr), :],
                bufs[i % 2].at[pl.ds(0, cr), :],
                sem_in[i % 2],
            )

        def start_outs(i):
            cr = _CHUNKS[i]
            return [
                pltpu.async_copy(
                    bufs[i % 2].at[pl.ds(0, cr), :],
                    out_hbm.at[b, pl.ds(base + _OFFS[i], cr), :],
                    sem_out[i % 2],
                )
                for b in range(_B)
            ]

        in_h = start_in(0)
        out_hs = {}
        for i in range(_N_CHUNKS):
            buf = bufs[i % 2]
            in_h.wait()
            if i >= 1:
                for h in out_hs.pop(i - 1):
                    h.wait()
            if i + 1 < _N_CHUNKS:
                in_h = start_in(i + 1)

            def scale_row(r, c, buf=buf):
                def scale_vec(j, c2, buf=buf, r=r):
                    sl = pl.ds(j * _L, _L)
                    buf[r, sl] = buf[r, sl] * _SCALE
                    return c2

                return lax.fori_loop(0, _DIM // _L, scale_vec, c, unroll=8)

            lax.fori_loop(0, _CHUNKS[i], scale_row, 0)
            out_hs[i] = start_outs(i)
        for h in out_hs.pop(_N_CHUNKS - 1):
            h.wait()

    return sc_broadcast


_sc_broadcast = _make_sc_broadcast()


def kernel(x, table):
    del x
    return _sc_broadcast(table)


# speedup vs baseline: 1.0592x; 1.0592x over previous
import functools

import jax
import jax.numpy as jnp
from jax import lax
from jax.experimental import pallas as pl
from jax.experimental.pallas import tpu as pltpu
from jax.experimental.pallas import tpu_sc as plsc

_DIM = 1024
_B = 4
_S = 8192
_SCALE = float(_DIM) ** (-0.5)

_NC = 2
_NS = 16
_NW = _NC * _NS
_L = 16

_ROWS_PER_W = _S // _NW
_CHUNKS = (32,) * 8
_MAX_CHUNK = max(_CHUNKS)
_OFFS = tuple(sum(_CHUNKS[:k]) for k in range(len(_CHUNKS)))
_N_CHUNKS = len(_CHUNKS)


def _make_sc_broadcast():
    mesh = plsc.VectorSubcoreMesh(core_axis_name="c", subcore_axis_name="s")

    @functools.partial(
        pl.kernel,
        mesh=mesh,
        out_type=jax.ShapeDtypeStruct((_B, _S, _DIM), jnp.float32),
        scratch_types=[
            pltpu.VMEM((_MAX_CHUNK, _DIM), jnp.float32),
            pltpu.VMEM((_MAX_CHUNK, _DIM), jnp.float32),
            pltpu.SemaphoreType.DMA,
            pltpu.SemaphoreType.DMA,
            pltpu.SemaphoreType.DMA,
            pltpu.SemaphoreType.DMA,
        ],
    )
    def sc_broadcast(table_hbm, out_hbm, buf0, buf1, si0, si1, so0, so1):
        bufs = (buf0, buf1)
        sem_in = (si0, si1)
        sem_out = (so0, so1)
        wid = lax.axis_index("s") * _NC + lax.axis_index("c")
        base = wid * _ROWS_PER_W

        def start_in(i):
            cr = _CHUNKS[i]
            return pltpu.async_copy(
                table_hbm.at[pl.ds(base + _OFFS[i], cr), :],
                bufs[i % 2].at[pl.ds(0, cr), :],
                sem_in[i % 2],
            )

        def start_outs(i):
            cr = _CHUNKS[i]
            return [
                pltpu.async_copy(
                    bufs[i % 2].at[pl.ds(0, cr), :],
                    out_hbm.at[b, pl.ds(base + _OFFS[i], cr), :],
                    sem_out[i % 2],
                )
                for b in range(_B)
            ]

        in_h = start_in(0)
        out_hs = {}
        for i in range(_N_CHUNKS):
            buf = bufs[i % 2]
            in_h.wait()
            if i >= 1:
                for h in out_hs.pop(i - 1):
                    h.wait()
            if i + 1 < _N_CHUNKS:
                in_h = start_in(i + 1)

            def scale_row(r, c, buf=buf):
                def scale_vec(j, c2, buf=buf, r=r):
                    sl = pl.ds(j * _L, _L)
                    buf[r, sl] = buf[r, sl] * _SCALE
                    return c2

                return lax.fori_loop(0, _DIM // _L, scale_vec, c, unroll=8)

            lax.fori_loop(0, _CHUNKS[i], scale_row, 0)
            out_hs[i] = start_outs(i)
        for h in out_hs.pop(_N_CHUNKS - 1):
            h.wait()

    return sc_broadcast


_sc_broadcast = _make_sc_broadcast()


def kernel(x, table):
    del x
    return _sc_broadcast(table)
